# baseline (speedup 1.0000x reference)
import functools

import jax
import jax.numpy as jnp
from jax.experimental import pallas as pl
from jax.experimental.pallas import tpu as pltpu


def _leaky(v):
    return jnp.where(v >= 0, v, 0.2 * v)


def _layer_body(bm, last,
                adj_ref, xk_ref, wg1_ref, wg2_ref, wb1_ref, wb2_ref,
                r_ref, wgc_ref, head_ref, *refs):
    if last:
        h_ref, logp_ref, out_ref = refs
    else:
        h_ref, out_ref = refs
    m_idx = pl.program_id(0)
    adj = adj_ref[...]
    s_mat = jax.lax.dot_general(adj, xk_ref[...], (((1,), (0,)), ((), ())),
                                preferred_element_type=jnp.float32)
    s = jnp.sum(adj, axis=1, keepdims=True)
    x = xk_ref[pl.ds(m_idx * bm, bm), :]
    r = r_ref[0:1, :]
    is_head = head_ref[0, 0] != 0.0
    neighbor = s_mat / jnp.maximum(s, 1e-12)
    gamma = _leaky(x @ wg1_ref[...] + neighbor @ wg2_ref[...]) + 1.0
    beta = _leaky(x @ wb1_ref[...] + neighbor @ wb2_ref[...])
    out = x + (gamma * r + beta) - neighbor
    wgc = wgc_ref[...]
    p = (s_mat + x) @ wgc
    h_head = p / jnp.maximum(s + 1.0, 1e-12)
    h_tail = (p + out @ wgc) / (s + 2.0)
    h = jnp.where(is_head, h_head, h_tail)
    out_ref[...] = out
    if last:
        h_ref[...] = h
        m = jnp.max(h, axis=1, keepdims=True)
        sh = h - m
        logp_ref[...] = sh - jnp.log(jnp.sum(jnp.exp(sh), axis=1, keepdims=True))
    else:
        h_ref[...] = jnp.where(h > 0, h, jnp.exp(h) - 1.0)


def _layer(xin, adj, wg1, wg2, wb1, wb2, r8, wgc, head8, last):
    n, f = xin.shape
    hid = wgc.shape[1]
    bm = 200
    mb = n // bm
    if last:
        out_shape = (jax.ShapeDtypeStruct((n, hid), jnp.float32),
                     jax.ShapeDtypeStruct((n, hid), jnp.float32),
                     jax.ShapeDtypeStruct((n, f), jnp.float32))
        out_specs = (pl.BlockSpec((bm, hid), lambda m: (m, 0)),
                     pl.BlockSpec((bm, hid), lambda m: (m, 0)),
                     pl.BlockSpec((bm, f), lambda m: (m, 0)))
    else:
        out_shape = (jax.ShapeDtypeStruct((n, hid), jnp.float32),
                     jax.ShapeDtypeStruct((n, f), jnp.float32))
        out_specs = (pl.BlockSpec((bm, hid), lambda m: (m, 0)),
                     pl.BlockSpec((bm, f), lambda m: (m, 0)))
    vmem = pl.BlockSpec(memory_space=pltpu.VMEM)
    return pl.pallas_call(
        functools.partial(_layer_body, bm, last),
        grid=(mb,),
        in_specs=[
            pl.BlockSpec((bm, n), lambda m: (m, 0)),
            vmem,
            vmem, vmem, vmem, vmem,
            vmem,
            vmem,
            vmem,
        ],
        out_specs=out_specs,
        out_shape=out_shape,
        compiler_params=pltpu.CompilerParams(
            dimension_semantics=("arbitrary",)),
    )(adj, xin, wg1, wg2, wb1, wb2, r8, wgc, head8)


def kernel(x, adj, Wg1a, Wg2a, Wb1a, Wb2a, ra, Wgca, Wg1b, Wg2b, Wb1b, Wb2b, rb, Wgcb, head):
    head8 = jnp.broadcast_to(
        jnp.asarray(head, jnp.float32).reshape(1, 1), (8, 128))
    ra8 = jnp.broadcast_to(ra, (8, ra.shape[1]))
    rb8 = jnp.broadcast_to(rb, (8, rb.shape[1]))
    x1, out1 = _layer(x, adj, Wg1a, Wg2a, Wb1a, Wb2a, ra8, Wgca, head8, False)
    x2, logp, out2 = _layer(x1, adj, Wg1b, Wg2b, Wb1b, Wb2b, rb8, Wgcb, head8, True)
    return (x2, logp, out1, out2)


# speedup vs baseline: 1.1155x; 1.1155x over previous
import functools

import jax
import jax.numpy as jnp
from jax.experimental import pallas as pl
from jax.experimental.pallas import tpu as pltpu


def _leaky(v):
    return jnp.where(v >= 0, v, 0.2 * v)


def _layer_body(bm, last,
                adj_ref, xk_ref, wg1_ref, wg2_ref, wb1_ref, wb2_ref,
                r_ref, wgc_ref, head_ref, *refs):
    if last:
        h_ref, logp_ref, out_ref = refs
    else:
        h_ref, out_ref = refs
    m_idx = pl.program_id(0)
    adj = adj_ref[...]
    s_mat = jax.lax.dot_general(adj, xk_ref[...], (((1,), (0,)), ((), ())),
                                preferred_element_type=jnp.float32)
    s = jnp.sum(adj, axis=1, keepdims=True)
    x = xk_ref[pl.ds(m_idx * bm, bm), :]
    r = r_ref[0:1, :]
    is_head = head_ref[0, 0] != 0.0
    neighbor = s_mat / jnp.maximum(s, 1e-12)
    gamma = _leaky(x @ wg1_ref[...] + neighbor @ wg2_ref[...]) + 1.0
    beta = _leaky(x @ wb1_ref[...] + neighbor @ wb2_ref[...])
    out = x + (gamma * r + beta) - neighbor
    wgc = wgc_ref[...]
    p = (s_mat + x) @ wgc
    h_head = p / jnp.maximum(s + 1.0, 1e-12)
    h_tail = (p + out @ wgc) / (s + 2.0)
    h = jnp.where(is_head, h_head, h_tail)
    out_ref[...] = out
    if last:
        h_ref[...] = h
        m = jnp.max(h, axis=1, keepdims=True)
        sh = h - m
        logp_ref[...] = sh - jnp.log(jnp.sum(jnp.exp(sh), axis=1, keepdims=True))
    else:
        h_ref[...] = jnp.where(h > 0, h, jnp.exp(h) - 1.0)


def _layer(xin, adj, wg1, wg2, wb1, wb2, r8, wgc, head8, last):
    n, f = xin.shape
    hid = wgc.shape[1]
    bm = 400
    mb = n // bm
    if last:
        out_shape = (jax.ShapeDtypeStruct((n, hid), jnp.float32),
                     jax.ShapeDtypeStruct((n, hid), jnp.float32),
                     jax.ShapeDtypeStruct((n, f), jnp.float32))
        out_specs = (pl.BlockSpec((bm, hid), lambda m: (m, 0)),
                     pl.BlockSpec((bm, hid), lambda m: (m, 0)),
                     pl.BlockSpec((bm, f), lambda m: (m, 0)))
    else:
        out_shape = (jax.ShapeDtypeStruct((n, hid), jnp.float32),
                     jax.ShapeDtypeStruct((n, f), jnp.float32))
        out_specs = (pl.BlockSpec((bm, hid), lambda m: (m, 0)),
                     pl.BlockSpec((bm, f), lambda m: (m, 0)))
    vmem = pl.BlockSpec(memory_space=pltpu.VMEM)
    return pl.pallas_call(
        functools.partial(_layer_body, bm, last),
        grid=(mb,),
        in_specs=[
            pl.BlockSpec((bm, n), lambda m: (m, 0)),
            vmem,
            vmem, vmem, vmem, vmem,
            vmem,
            vmem,
            vmem,
        ],
        out_specs=out_specs,
        out_shape=out_shape,
        compiler_params=pltpu.CompilerParams(
            dimension_semantics=("parallel",)),
    )(adj, xin, wg1, wg2, wb1, wb2, r8, wgc, head8)


def kernel(x, adj, Wg1a, Wg2a, Wb1a, Wb2a, ra, Wgca, Wg1b, Wg2b, Wb1b, Wb2b, rb, Wgcb, head):
    head8 = jnp.broadcast_to(
        jnp.asarray(head, jnp.float32).reshape(1, 1), (8, 128))
    ra8 = jnp.broadcast_to(ra, (8, ra.shape[1]))
    rb8 = jnp.broadcast_to(rb, (8, rb.shape[1]))
    x1, out1 = _layer(x, adj, Wg1a, Wg2a, Wb1a, Wb2a, ra8, Wgca, head8, False)
    x2, logp, out2 = _layer(x1, adj, Wg1b, Wg2b, Wb1b, Wb2b, rb8, Wgcb, head8, True)
    return (x2, logp, out1, out2)
